# Initial kernel scaffold; baseline (speedup 1.0000x reference)
import jax
import jax.numpy as jnp
from jax.experimental import pallas as pl


def kernel(x, emb):
    raise NotImplementedError("write your pallas kernel here")


# speedup vs baseline: 1.2525x; 1.2525x over previous
import jax
import jax.numpy as jnp
from jax.experimental import pallas as pl
from jax.experimental.pallas import tpu as pltpu

_BN = 64


def _add_body(x_ref, e_ref, o_ref):
    o_ref[...] = x_ref[...] + e_ref[...][None, :, None, :]


def kernel(x, emb):
    B, N, T, D = x.shape
    bn = _BN if N % _BN == 0 else N
    grid = (B, N // bn)
    return pl.pallas_call(
        _add_body,
        grid=grid,
        in_specs=[
            pl.BlockSpec((1, bn, T, D), lambda i, j: (i, j, 0, 0)),
            pl.BlockSpec((bn, D), lambda i, j: (j, 0)),
        ],
        out_specs=pl.BlockSpec((1, bn, T, D), lambda i, j: (i, j, 0, 0)),
        out_shape=jax.ShapeDtypeStruct(x.shape, x.dtype),
    )(x, emb)


# speedup vs baseline: 1.2821x; 1.0237x over previous
import jax
import jax.numpy as jnp
from jax.experimental import pallas as pl
from jax.experimental.pallas import tpu as pltpu

_BN = 64


def _add_body(x_ref, e_ref, o_ref):
    o_ref[...] = x_ref[...] + e_ref[...][None, :, None, :]


def kernel(x, emb):
    B, N, T, D = x.shape
    bn = _BN if N % _BN == 0 else N
    grid = (N // bn, B)
    return pl.pallas_call(
        _add_body,
        grid=grid,
        in_specs=[
            pl.BlockSpec((1, bn, T, D), lambda j, i: (i, j, 0, 0)),
            pl.BlockSpec((bn, D), lambda j, i: (j, 0)),
        ],
        out_specs=pl.BlockSpec((1, bn, T, D), lambda j, i: (i, j, 0, 0)),
        out_shape=jax.ShapeDtypeStruct(x.shape, x.dtype),
    )(x, emb)


# speedup vs baseline: 1.4112x; 1.1007x over previous
import jax
import jax.numpy as jnp
from jax.experimental import pallas as pl
from jax.experimental.pallas import tpu as pltpu

_BN = 128


def _add_body(x_ref, e_ref, o_ref):
    o_ref[...] = x_ref[...] + e_ref[...][None, :, None, :]


def kernel(x, emb):
    B, N, T, D = x.shape
    bn = _BN if N % _BN == 0 else N
    grid = (N // bn, B)
    return pl.pallas_call(
        _add_body,
        grid=grid,
        in_specs=[
            pl.BlockSpec((1, bn, T, D), lambda j, i: (i, j, 0, 0)),
            pl.BlockSpec((bn, D), lambda j, i: (j, 0)),
        ],
        out_specs=pl.BlockSpec((1, bn, T, D), lambda j, i: (i, j, 0, 0)),
        out_shape=jax.ShapeDtypeStruct(x.shape, x.dtype),
    )(x, emb)


# speedup vs baseline: 1.4599x; 1.0345x over previous
import jax
import jax.numpy as jnp
from jax.experimental import pallas as pl
from jax.experimental.pallas import tpu as pltpu

_BN = 256


def _add_body(x_ref, e_ref, o_ref):
    o_ref[...] = x_ref[...] + e_ref[...][None, :, None, :]


def kernel(x, emb):
    B, N, T, D = x.shape
    bn = _BN if N % _BN == 0 else N
    grid = (N // bn, B)
    return pl.pallas_call(
        _add_body,
        grid=grid,
        in_specs=[
            pl.BlockSpec((1, bn, T, D), lambda j, i: (i, j, 0, 0)),
            pl.BlockSpec((bn, D), lambda j, i: (j, 0)),
        ],
        out_specs=pl.BlockSpec((1, bn, T, D), lambda j, i: (i, j, 0, 0)),
        out_shape=jax.ShapeDtypeStruct(x.shape, x.dtype),
    )(x, emb)
